# Initial kernel scaffold; baseline (speedup 1.0000x reference)
import jax
import jax.numpy as jnp
from jax.experimental import pallas as pl


def kernel(h_0, E_attr, m_W1, m_b1, m_W2, m_b2, gru_Wih, gru_Whh, gru_bih, gru_bhh, ri_W1, ri_b1, ri_W2, ri_b2, rj_W1, rj_b1, rj_W2, rj_b2, c_W1, c_b1, c_W2, c_b2, graph_index, E):
    raise NotImplementedError("write your pallas kernel here")


# speedup vs baseline: 1.0854x; 1.0854x over previous
import functools

import jax
import jax.numpy as jnp
from jax import lax
from jax.experimental import pallas as pl
from jax.experimental.pallas import tpu as pltpu
from jax.experimental.pallas import tpu_sc as plsc

N_NODE = 10000
N_EDGE = 320000
N_DIM = 128
E_DIM = 16
M_DIM = 128
G_DIM = 128
T = 3
N_GRAPH = 64

NP = 10240
EP = 327680
NS = 16
NC = 2
C = 80
EPT = EP // NS
NCH = EPT // C
RPT = NP // NS

_f32 = jnp.float32


def _sc_deg_body(srcp, deg_out, src_v, ones_v, zz_v, deg_sh, sem):
    c = lax.axis_index("c")
    s = lax.axis_index("s")
    del sem

    for i in range(16):
        for k in range(8):
            zz_v[i, pl.ds(k * 16, 16)] = jnp.zeros((16,), _f32)
    for i in range(C):
        for k in range(8):
            ones_v[i, pl.ds(k * 16, 16)] = jnp.ones((16,), _f32)

    def _zero_d(j, _):
        pltpu.sync_copy(zz_v, deg_sh.at[pl.ds(s * RPT + j * 16, 16)])
        return 0
    lax.fori_loop(0, RPT // 16, _zero_d, 0)
    plsc.subcore_barrier()

    def _chunk(i, _):
        base = s * EPT + i * C
        pltpu.sync_copy(srcp.at[pl.ds(base, C)], src_v)
        pltpu.sync_copy(ones_v, deg_sh.at[src_v], add=True)
        return 0
    lax.fori_loop(0, NCH, _chunk, 0)
    plsc.subcore_barrier()

    @pl.when(c == 0)
    def _():
        pltpu.sync_copy(deg_sh.at[pl.ds(s * RPT, RPT)],
                        deg_out.at[pl.ds(s * RPT, RPT)])


@functools.cache
def _sc_deg_fn():
    return pl.kernel(
        _sc_deg_body,
        out_type=[jax.ShapeDtypeStruct((NP, 128), _f32)],
        mesh=plsc.VectorSubcoreMesh(core_axis_name="c", subcore_axis_name="s",
                                    num_cores=NC, num_subcores=NS),
        scratch_types=[
            pltpu.VMEM((C,), jnp.int32),
            pltpu.VMEM((C, 128), _f32),
            pltpu.VMEM((16, 128), _f32),
            pltpu.VMEM_SHARED((NP, 128), _f32),
            pltpu.SemaphoreType.DMA,
        ],
    )


def _sc_deg(srcp):
    return _sc_deg_fn()(srcp)[0]


def _sc_body(z2, be2, srcp, dstp, s2_out,
             dst_v, src_v, zbuf, bbuf, zrow_v, s_sh, sem):
    c = lax.axis_index("c")
    s = lax.axis_index("s")

    for r in range(16):
        for k in range(8):
            zrow_v[r, pl.ds(k * 16, 16)] = jnp.zeros((16,), _f32)

    def _zero_s(j, _):
        pltpu.sync_copy(zrow_v, s_sh.at[pl.ds(s * RPT + j * 16, 16)])
        return 0
    lax.fori_loop(0, RPT // 16, _zero_s, 0)
    plsc.subcore_barrier()

    def _chunk(i, _):
        base = s * EPT + i * C
        pltpu.sync_copy(dstp.at[pl.ds(base, C)], dst_v)
        pltpu.sync_copy(srcp.at[pl.ds(base, C)], src_v)
        pltpu.async_copy(z2.at[c].at[dst_v], zbuf, sem).wait()
        pltpu.sync_copy(be2.at[c].at[pl.ds(base, C)], bbuf)

        def _row(r, _):
            for k in range(8):
                sl = pl.ds(k * 16, 16)
                zbuf[r, sl] = jnp.maximum(zbuf[r, sl] + bbuf[r, sl], 0.0)
            return 0
        lax.fori_loop(0, C, _row, 0)

        pltpu.sync_copy(zbuf, s_sh.at[src_v], add=True)
        return 0
    lax.fori_loop(0, NCH, _chunk, 0)
    plsc.subcore_barrier()

    pltpu.sync_copy(s_sh.at[pl.ds(s * RPT, RPT)],
                    s2_out.at[c].at[pl.ds(s * RPT, RPT)])


@functools.cache
def _sc_step_fn():
    return pl.kernel(
        _sc_body,
        out_type=[jax.ShapeDtypeStruct((NC, NP, 128), _f32)],
        mesh=plsc.VectorSubcoreMesh(core_axis_name="c", subcore_axis_name="s",
                                    num_cores=NC, num_subcores=NS),
        scratch_types=[
            pltpu.VMEM((C,), jnp.int32),
            pltpu.VMEM((C,), jnp.int32),
            pltpu.VMEM((C, 128), _f32),
            pltpu.VMEM((C, 128), _f32),
            pltpu.VMEM((16, 128), _f32),
            pltpu.VMEM_SHARED((NP, 128), _f32),
            pltpu.SemaphoreType.DMA,
        ],
    )


def _sc_step(z2, be2, srcp, dstp):
    return _sc_step_fn()(z2, be2, srcp, dstp)[0]


def _dot_t(a, b):
    return lax.dot_general(a, b, (((1,), (1,)), ((), ())),
                           preferred_element_type=_f32)


def _mm(a, b):
    return jnp.dot(a, b, preferred_element_type=_f32)


def _edge_bias_body(et_ref, w_ref, b_ref, out_ref):
    r = _mm(et_ref[...], w_ref[...]) + b_ref[...]
    out_ref[0] = r[:, :128]
    out_ref[1] = r[:, 128:]


def _edge_bias(e_t, w1e, b1):
    grid = EP // 2048
    return pl.pallas_call(
        _edge_bias_body,
        grid=(grid,),
        in_specs=[pl.BlockSpec((2048, E_DIM), lambda i: (i, 0)),
                  pl.BlockSpec((E_DIM, 256), lambda i: (0, 0)),
                  pl.BlockSpec((1, 256), lambda i: (0, 0))],
        out_specs=pl.BlockSpec((2, 2048, 128), lambda i: (0, i, 0)),
        out_shape=jax.ShapeDtypeStruct((2, EP, 128), _f32),
    )(e_t, w1e, b1)


def _z_proj_body(h_ref, w_ref, out_ref):
    z = _mm(h_ref[...], w_ref[...])
    out_ref[0] = z[:, :128]
    out_ref[1] = z[:, 128:]


def _z_proj(h, w1h):
    grid = NP // 1024
    return pl.pallas_call(
        _z_proj_body,
        grid=(grid,),
        in_specs=[pl.BlockSpec((1024, 128), lambda i: (i, 0)),
                  pl.BlockSpec((128, 256), lambda i: (0, 0))],
        out_specs=pl.BlockSpec((2, 1024, 128), lambda i: (0, i, 0)),
        out_shape=jax.ShapeDtypeStruct((2, NP, 128), _f32),
    )(h, w1h)


def _update_body(s2_ref, deg_ref, h_ref, w2a_ref, w2b_ref, b2_ref,
                 wih_ref, whh_ref, bih_ref, bhh_ref, w1h_ref,
                 h_out, z_out):
    m_v = (_mm(s2_ref[0], w2a_ref[...]) + _mm(s2_ref[1], w2b_ref[...])
           + deg_ref[...] * b2_ref[...])
    h = h_ref[...]
    gi = _dot_t(m_v, wih_ref[...]) + bih_ref[...]
    gh = _dot_t(h, whh_ref[...]) + bhh_ref[...]
    r = jax.nn.sigmoid(gi[:, :128] + gh[:, :128])
    z = jax.nn.sigmoid(gi[:, 128:256] + gh[:, 128:256])
    n = jnp.tanh(gi[:, 256:] + r * gh[:, 256:])
    h_new = (1.0 - z) * n + z * h
    h_out[...] = h_new
    zp = _mm(h_new, w1h_ref[...])
    z_out[0] = zp[:, :128]
    z_out[1] = zp[:, 128:]


def _update(s2, deg, h, w2a, w2b, b2, wih, whh, bih, bhh, w1h):
    grid = NP // 1024
    return pl.pallas_call(
        _update_body,
        grid=(grid,),
        in_specs=[pl.BlockSpec((2, 1024, 128), lambda i: (0, i, 0)),
                  pl.BlockSpec((1024, 1), lambda i: (i, 0)),
                  pl.BlockSpec((1024, 128), lambda i: (i, 0)),
                  pl.BlockSpec((128, 128), lambda i: (0, 0)),
                  pl.BlockSpec((128, 128), lambda i: (0, 0)),
                  pl.BlockSpec((1, 128), lambda i: (0, 0)),
                  pl.BlockSpec((384, 128), lambda i: (0, 0)),
                  pl.BlockSpec((384, 128), lambda i: (0, 0)),
                  pl.BlockSpec((1, 384), lambda i: (0, 0)),
                  pl.BlockSpec((1, 384), lambda i: (0, 0)),
                  pl.BlockSpec((128, 256), lambda i: (0, 0))],
        out_specs=[pl.BlockSpec((1024, 128), lambda i: (i, 0)),
                   pl.BlockSpec((2, 1024, 128), lambda i: (0, i, 0))],
        out_shape=[jax.ShapeDtypeStruct((NP, 128), _f32),
                   jax.ShapeDtypeStruct((2, NP, 128), _f32)],
    )(s2, deg, h, w2a, w2b, b2, wih, whh, bih, bhh, w1h)


def _readout_body(h_ref, h0_ref, gid_ref,
                  ria_ref, rib_ref, rb1_ref, riw2_ref, rb2_ref,
                  rjw1_ref, rjb1_ref, rjw2_ref, rjb2_ref,
                  cw1_ref, cb1_ref, cw2_ref, cb2_ref,
                  out_ref, racc):
    i = pl.program_id(0)

    @pl.when(i == 0)
    def _():
        racc[...] = jnp.zeros((N_GRAPH, G_DIM), _f32)

    h = h_ref[...]
    h0 = h0_ref[...]
    pre_i = jnp.maximum(_mm(h, ria_ref[...]) + _mm(h0, rib_ref[...])
                        + rb1_ref[...], 0.0)
    i_out = jax.nn.sigmoid(_mm(pre_i, riw2_ref[...]) + rb2_ref[...])
    pre_j = jnp.maximum(_mm(h, rjw1_ref[...]) + rjb1_ref[...], 0.0)
    j_out = _mm(pre_j, rjw2_ref[...]) + rjb2_ref[...]
    p = i_out * j_out

    gid = gid_ref[...]
    seg = lax.broadcasted_iota(jnp.int32, (N_GRAPH, 1024), 0)
    ht = (seg == gid).astype(_f32)
    racc[...] += lax.dot_general(ht, p, (((1,), (0,)), ((), ())),
                                 preferred_element_type=_f32)

    @pl.when(i == pl.num_programs(0) - 1)
    def _():
        rr = racc[...]
        l1 = jnp.maximum(_mm(rr, cw1_ref[...]) + cb1_ref[...], 0.0)
        logit = _mm(l1, cw2_ref[...]) + cb2_ref[...]
        out_ref[...] = jax.nn.sigmoid(logit)


def _readout(h, h0, gid, ria, rib, rb1, riw2, rb2,
             rjw1, rjb1, rjw2, rjb2, cw1, cb1, cw2, cb2):
    grid = NP // 1024
    return pl.pallas_call(
        _readout_body,
        grid=(grid,),
        in_specs=[pl.BlockSpec((1024, 128), lambda i: (i, 0)),
                  pl.BlockSpec((1024, 128), lambda i: (i, 0)),
                  pl.BlockSpec((1, 1024), lambda i: (0, i)),
                  pl.BlockSpec((128, 256), lambda i: (0, 0)),
                  pl.BlockSpec((128, 256), lambda i: (0, 0)),
                  pl.BlockSpec((1, 256), lambda i: (0, 0)),
                  pl.BlockSpec((256, 128), lambda i: (0, 0)),
                  pl.BlockSpec((1, 128), lambda i: (0, 0)),
                  pl.BlockSpec((128, 256), lambda i: (0, 0)),
                  pl.BlockSpec((1, 256), lambda i: (0, 0)),
                  pl.BlockSpec((256, 128), lambda i: (0, 0)),
                  pl.BlockSpec((1, 128), lambda i: (0, 0)),
                  pl.BlockSpec((128, 128), lambda i: (0, 0)),
                  pl.BlockSpec((1, 128), lambda i: (0, 0)),
                  pl.BlockSpec((128, 1), lambda i: (0, 0)),
                  pl.BlockSpec((1, 1), lambda i: (0, 0))],
        out_specs=pl.BlockSpec((N_GRAPH, 1), lambda i: (0, 0)),
        out_shape=jax.ShapeDtypeStruct((N_GRAPH, 1), _f32),
        scratch_shapes=[pltpu.VMEM((N_GRAPH, G_DIM), _f32)],
        compiler_params=pltpu.CompilerParams(
            dimension_semantics=("arbitrary",)),
    )(h, h0, gid, ria, rib, rb1, riw2, rb2,
      rjw1, rjb1, rjw2, rjb2, cw1, cb1, cw2, cb2)


def kernel(h_0, E_attr, m_W1, m_b1, m_W2, m_b2, gru_Wih, gru_Whh, gru_bih,
           gru_bhh, ri_W1, ri_b1, ri_W2, ri_b2, rj_W1, rj_b1, rj_W2, rj_b2,
           c_W1, c_b1, c_W2, c_b2, graph_index, E):
    w1h = m_W1[:N_DIM]
    w1e = m_W1[N_DIM:]
    w2a = m_W2[:128]
    w2b = m_W2[128:]
    b1 = m_b1.reshape(1, 256)
    b2 = m_b2.reshape(1, 128)
    bih = gru_bih.reshape(1, 384)
    bhh = gru_bhh.reshape(1, 384)
    ria = ri_W1[:N_DIM]
    rib = ri_W1[N_DIM:]
    rb1 = ri_b1.reshape(1, 256)
    rb2 = ri_b2.reshape(1, 128)
    rjb1 = rj_b1.reshape(1, 256)
    rjb2 = rj_b2.reshape(1, 128)
    cb1 = c_b1.reshape(1, 128)
    cb2 = c_b2.reshape(1, 1)

    e_t = jnp.pad(E_attr.T, ((0, EP - N_EDGE), (0, 0)))
    srcp = jnp.pad(E[0], (0, EP - N_EDGE), constant_values=N_NODE)
    dstp = jnp.pad(E[1], (0, EP - N_EDGE), constant_values=0)
    h0p = jnp.pad(h_0, ((0, NP - N_NODE), (0, 0)))
    gid = jnp.pad(graph_index, (0, NP - N_NODE),
                  constant_values=N_GRAPH).reshape(1, NP)

    be2 = _edge_bias(e_t, w1e, b1)

    deg = _sc_deg(srcp)[:, :1]
    h = h0p
    z2 = _z_proj(h, w1h)
    for _ in range(T):
        s2 = _sc_step(z2, be2, srcp, dstp)
        h, z2 = _update(s2, deg, h, w2a, w2b, b2,
                        gru_Wih, gru_Whh, bih, bhh, w1h)

    out = _readout(h, h0p, gid, ria, rib, rb1, ri_W2, rb2,
                   rj_W1, rjb1, rj_W2, rjb2, c_W1, cb1, c_W2, cb2)
    return out.reshape(N_GRAPH)


# speedup vs baseline: 1.7820x; 1.6418x over previous
import functools

import jax
import jax.numpy as jnp
from jax import lax
from jax.experimental import pallas as pl
from jax.experimental.pallas import tpu as pltpu
from jax.experimental.pallas import tpu_sc as plsc

N_NODE = 10000
N_EDGE = 320000
N_DIM = 128
E_DIM = 16
M_DIM = 128
G_DIM = 128
T = 3
N_GRAPH = 64

NP = 10240
EP = 327680
NS = 16
NC = 2
C = 40
EPT = EP // NS
NCH = EPT // C
RPT = NP // NS

_f32 = jnp.float32


def _sc_deg_body(srcp, deg_out, src_v, ones_v, zz_v, deg_sh, sem):
    c = lax.axis_index("c")
    s = lax.axis_index("s")
    del sem

    for i in range(16):
        for k in range(8):
            zz_v[i, pl.ds(k * 16, 16)] = jnp.zeros((16,), _f32)
    for i in range(C):
        for k in range(8):
            ones_v[i, pl.ds(k * 16, 16)] = jnp.ones((16,), _f32)

    def _zero_d(j, _):
        pltpu.sync_copy(zz_v, deg_sh.at[pl.ds(s * RPT + j * 16, 16)])
        return 0
    lax.fori_loop(0, RPT // 16, _zero_d, 0)
    plsc.subcore_barrier()

    def _chunk(i, _):
        base = s * EPT + i * C
        pltpu.sync_copy(srcp.at[pl.ds(base, C)], src_v)
        pltpu.sync_copy(ones_v, deg_sh.at[src_v], add=True)
        return 0
    lax.fori_loop(0, NCH, _chunk, 0)
    plsc.subcore_barrier()

    @pl.when(c == 0)
    def _():
        pltpu.sync_copy(deg_sh.at[pl.ds(s * RPT, RPT)],
                        deg_out.at[pl.ds(s * RPT, RPT)])


@functools.cache
def _sc_deg_fn():
    return pl.kernel(
        _sc_deg_body,
        out_type=[jax.ShapeDtypeStruct((NP, 128), _f32)],
        mesh=plsc.VectorSubcoreMesh(core_axis_name="c", subcore_axis_name="s",
                                    num_cores=NC, num_subcores=NS),
        scratch_types=[
            pltpu.VMEM((C,), jnp.int32),
            pltpu.VMEM((C, 128), _f32),
            pltpu.VMEM((16, 128), _f32),
            pltpu.VMEM_SHARED((NP, 128), _f32),
            pltpu.SemaphoreType.DMA,
        ],
    )


def _sc_deg(srcp):
    return _sc_deg_fn()(srcp)[0]


def _sc_body(z2, be2, srcp, dstp, s2_out,
             dst_v, src_v, zbuf, bbuf, zrow_v, s_sh,
             gsem0, gsem1, ssem0, ssem1, dsem0, dsem1):
    c = lax.axis_index("c")
    s = lax.axis_index("s")
    gsem = (gsem0, gsem1)
    ssem = (ssem0, ssem1)
    dsem = (dsem0, dsem1)

    for r in range(16):
        for k in range(8):
            zrow_v[r, pl.ds(k * 16, 16)] = jnp.zeros((16,), _f32)

    def _zero_s(j, _):
        pltpu.sync_copy(zrow_v, s_sh.at[pl.ds(s * RPT + j * 16, 16)])
        return 0
    lax.fori_loop(0, RPT // 16, _zero_s, 0)
    plsc.subcore_barrier()

    def _issue_gather(k, b):
        base = s * EPT + k * C
        pltpu.async_copy(z2.at[c].at[dst_v.at[b]], zbuf.at[b], gsem[b])
        pltpu.async_copy(be2.at[c].at[pl.ds(base, C)], bbuf.at[b], gsem[b])

    def _wait_gather(b):
        pltpu.make_async_copy(z2.at[c].at[dst_v.at[b]], zbuf.at[b],
                              gsem[b]).wait()
        pltpu.make_async_copy(be2.at[c].at[pl.ds(0, C)], bbuf.at[b],
                              gsem[b]).wait()

    for b in range(2):
        base = s * EPT + b * C
        pltpu.sync_copy(dstp.at[pl.ds(base, C)], dst_v.at[b])
        _issue_gather(b, b)
        pltpu.async_copy(srcp.at[pl.ds(base, C)], src_v.at[b], ssem[b])

    def _pair(i, _):
        for b in range(2):
            k = i * 2 + b
            nk = k + 2
            _wait_gather(b)
            @pl.when(nk < NCH)
            def _():
                nbase = s * EPT + nk * C
                pltpu.async_copy(dstp.at[pl.ds(nbase, C)], dst_v.at[b],
                                 dsem[b])
            pltpu.make_async_copy(srcp.at[pl.ds(0, C)], src_v.at[b],
                                  ssem[b]).wait()

            @plsc.parallel_loop(0, C, 1, unroll=8)
            def _row(r):
                for kk in range(8):
                    sl = pl.ds(kk * 16, 16)
                    zbuf[b, r, sl] = jnp.maximum(
                        zbuf[b, r, sl] + bbuf[b, r, sl], 0.0)

            pltpu.sync_copy(zbuf.at[b], s_sh.at[src_v.at[b]], add=True)

            @pl.when(nk < NCH)
            def _():
                nbase = s * EPT + nk * C
                pltpu.async_copy(srcp.at[pl.ds(nbase, C)], src_v.at[b],
                                 ssem[b])
                pltpu.make_async_copy(dstp.at[pl.ds(0, C)], dst_v.at[b],
                                      dsem[b]).wait()
                _issue_gather(nk, b)
        return 0
    lax.fori_loop(0, NCH // 2, _pair, 0)
    plsc.subcore_barrier()

    pltpu.sync_copy(s_sh.at[pl.ds(s * RPT, RPT)],
                    s2_out.at[c].at[pl.ds(s * RPT, RPT)])


@functools.cache
def _sc_step_fn():
    return pl.kernel(
        _sc_body,
        out_type=[jax.ShapeDtypeStruct((NC, NP, 128), _f32)],
        mesh=plsc.VectorSubcoreMesh(core_axis_name="c", subcore_axis_name="s",
                                    num_cores=NC, num_subcores=NS),
        scratch_types=[
            pltpu.VMEM((2, C), jnp.int32),
            pltpu.VMEM((2, C), jnp.int32),
            pltpu.VMEM((2, C, 128), _f32),
            pltpu.VMEM((2, C, 128), _f32),
            pltpu.VMEM((16, 128), _f32),
            pltpu.VMEM_SHARED((NP, 128), _f32),
            pltpu.SemaphoreType.DMA,
            pltpu.SemaphoreType.DMA,
            pltpu.SemaphoreType.DMA,
            pltpu.SemaphoreType.DMA,
            pltpu.SemaphoreType.DMA,
            pltpu.SemaphoreType.DMA,
        ],
    )


def _sc_step(z2, be2, srcp, dstp):
    return _sc_step_fn()(z2, be2, srcp, dstp)[0]


def _dot_t(a, b):
    return lax.dot_general(a, b, (((1,), (1,)), ((), ())),
                           preferred_element_type=_f32)


def _mm(a, b):
    return jnp.dot(a, b, preferred_element_type=_f32)


def _edge_bias_body(et_ref, w_ref, b_ref, out_ref):
    r = _mm(et_ref[...], w_ref[...]) + b_ref[...]
    out_ref[0] = r[:, :128]
    out_ref[1] = r[:, 128:]


def _edge_bias(e_t, w1e, b1):
    grid = EP // 2048
    return pl.pallas_call(
        _edge_bias_body,
        grid=(grid,),
        in_specs=[pl.BlockSpec((2048, E_DIM), lambda i: (i, 0)),
                  pl.BlockSpec((E_DIM, 256), lambda i: (0, 0)),
                  pl.BlockSpec((1, 256), lambda i: (0, 0))],
        out_specs=pl.BlockSpec((2, 2048, 128), lambda i: (0, i, 0)),
        out_shape=jax.ShapeDtypeStruct((2, EP, 128), _f32),
    )(e_t, w1e, b1)


def _z_proj_body(h_ref, w_ref, out_ref):
    z = _mm(h_ref[...], w_ref[...])
    out_ref[0] = z[:, :128]
    out_ref[1] = z[:, 128:]


def _z_proj(h, w1h):
    grid = NP // 1024
    return pl.pallas_call(
        _z_proj_body,
        grid=(grid,),
        in_specs=[pl.BlockSpec((1024, 128), lambda i: (i, 0)),
                  pl.BlockSpec((128, 256), lambda i: (0, 0))],
        out_specs=pl.BlockSpec((2, 1024, 128), lambda i: (0, i, 0)),
        out_shape=jax.ShapeDtypeStruct((2, NP, 128), _f32),
    )(h, w1h)


def _update_body(s2_ref, deg_ref, h_ref, w2a_ref, w2b_ref, b2_ref,
                 wih_ref, whh_ref, bih_ref, bhh_ref, w1h_ref,
                 h_out, z_out):
    m_v = (_mm(s2_ref[0], w2a_ref[...]) + _mm(s2_ref[1], w2b_ref[...])
           + deg_ref[...] * b2_ref[...])
    h = h_ref[...]
    gi = _dot_t(m_v, wih_ref[...]) + bih_ref[...]
    gh = _dot_t(h, whh_ref[...]) + bhh_ref[...]
    r = jax.nn.sigmoid(gi[:, :128] + gh[:, :128])
    z = jax.nn.sigmoid(gi[:, 128:256] + gh[:, 128:256])
    n = jnp.tanh(gi[:, 256:] + r * gh[:, 256:])
    h_new = (1.0 - z) * n + z * h
    h_out[...] = h_new
    zp = _mm(h_new, w1h_ref[...])
    z_out[0] = zp[:, :128]
    z_out[1] = zp[:, 128:]


def _update(s2, deg, h, w2a, w2b, b2, wih, whh, bih, bhh, w1h):
    grid = NP // 1024
    return pl.pallas_call(
        _update_body,
        grid=(grid,),
        in_specs=[pl.BlockSpec((2, 1024, 128), lambda i: (0, i, 0)),
                  pl.BlockSpec((1024, 1), lambda i: (i, 0)),
                  pl.BlockSpec((1024, 128), lambda i: (i, 0)),
                  pl.BlockSpec((128, 128), lambda i: (0, 0)),
                  pl.BlockSpec((128, 128), lambda i: (0, 0)),
                  pl.BlockSpec((1, 128), lambda i: (0, 0)),
                  pl.BlockSpec((384, 128), lambda i: (0, 0)),
                  pl.BlockSpec((384, 128), lambda i: (0, 0)),
                  pl.BlockSpec((1, 384), lambda i: (0, 0)),
                  pl.BlockSpec((1, 384), lambda i: (0, 0)),
                  pl.BlockSpec((128, 256), lambda i: (0, 0))],
        out_specs=[pl.BlockSpec((1024, 128), lambda i: (i, 0)),
                   pl.BlockSpec((2, 1024, 128), lambda i: (0, i, 0))],
        out_shape=[jax.ShapeDtypeStruct((NP, 128), _f32),
                   jax.ShapeDtypeStruct((2, NP, 128), _f32)],
    )(s2, deg, h, w2a, w2b, b2, wih, whh, bih, bhh, w1h)


def _readout_body(h_ref, h0_ref, gid_ref,
                  ria_ref, rib_ref, rb1_ref, riw2_ref, rb2_ref,
                  rjw1_ref, rjb1_ref, rjw2_ref, rjb2_ref,
                  cw1_ref, cb1_ref, cw2_ref, cb2_ref,
                  out_ref, racc):
    i = pl.program_id(0)

    @pl.when(i == 0)
    def _():
        racc[...] = jnp.zeros((N_GRAPH, G_DIM), _f32)

    h = h_ref[...]
    h0 = h0_ref[...]
    pre_i = jnp.maximum(_mm(h, ria_ref[...]) + _mm(h0, rib_ref[...])
                        + rb1_ref[...], 0.0)
    i_out = jax.nn.sigmoid(_mm(pre_i, riw2_ref[...]) + rb2_ref[...])
    pre_j = jnp.maximum(_mm(h, rjw1_ref[...]) + rjb1_ref[...], 0.0)
    j_out = _mm(pre_j, rjw2_ref[...]) + rjb2_ref[...]
    p = i_out * j_out

    gid = gid_ref[...]
    seg = lax.broadcasted_iota(jnp.int32, (N_GRAPH, 1024), 0)
    ht = (seg == gid).astype(_f32)
    racc[...] += lax.dot_general(ht, p, (((1,), (0,)), ((), ())),
                                 preferred_element_type=_f32)

    @pl.when(i == pl.num_programs(0) - 1)
    def _():
        rr = racc[...]
        l1 = jnp.maximum(_mm(rr, cw1_ref[...]) + cb1_ref[...], 0.0)
        logit = _mm(l1, cw2_ref[...]) + cb2_ref[...]
        out_ref[...] = jax.nn.sigmoid(logit)


def _readout(h, h0, gid, ria, rib, rb1, riw2, rb2,
             rjw1, rjb1, rjw2, rjb2, cw1, cb1, cw2, cb2):
    grid = NP // 1024
    return pl.pallas_call(
        _readout_body,
        grid=(grid,),
        in_specs=[pl.BlockSpec((1024, 128), lambda i: (i, 0)),
                  pl.BlockSpec((1024, 128), lambda i: (i, 0)),
                  pl.BlockSpec((1, 1024), lambda i: (0, i)),
                  pl.BlockSpec((128, 256), lambda i: (0, 0)),
                  pl.BlockSpec((128, 256), lambda i: (0, 0)),
                  pl.BlockSpec((1, 256), lambda i: (0, 0)),
                  pl.BlockSpec((256, 128), lambda i: (0, 0)),
                  pl.BlockSpec((1, 128), lambda i: (0, 0)),
                  pl.BlockSpec((128, 256), lambda i: (0, 0)),
                  pl.BlockSpec((1, 256), lambda i: (0, 0)),
                  pl.BlockSpec((256, 128), lambda i: (0, 0)),
                  pl.BlockSpec((1, 128), lambda i: (0, 0)),
                  pl.BlockSpec((128, 128), lambda i: (0, 0)),
                  pl.BlockSpec((1, 128), lambda i: (0, 0)),
                  pl.BlockSpec((128, 1), lambda i: (0, 0)),
                  pl.BlockSpec((1, 1), lambda i: (0, 0))],
        out_specs=pl.BlockSpec((N_GRAPH, 1), lambda i: (0, 0)),
        out_shape=jax.ShapeDtypeStruct((N_GRAPH, 1), _f32),
        scratch_shapes=[pltpu.VMEM((N_GRAPH, G_DIM), _f32)],
        compiler_params=pltpu.CompilerParams(
            dimension_semantics=("arbitrary",)),
    )(h, h0, gid, ria, rib, rb1, riw2, rb2,
      rjw1, rjb1, rjw2, rjb2, cw1, cb1, cw2, cb2)


def kernel(h_0, E_attr, m_W1, m_b1, m_W2, m_b2, gru_Wih, gru_Whh, gru_bih,
           gru_bhh, ri_W1, ri_b1, ri_W2, ri_b2, rj_W1, rj_b1, rj_W2, rj_b2,
           c_W1, c_b1, c_W2, c_b2, graph_index, E):
    w1h = m_W1[:N_DIM]
    w1e = m_W1[N_DIM:]
    w2a = m_W2[:128]
    w2b = m_W2[128:]
    b1 = m_b1.reshape(1, 256)
    b2 = m_b2.reshape(1, 128)
    bih = gru_bih.reshape(1, 384)
    bhh = gru_bhh.reshape(1, 384)
    ria = ri_W1[:N_DIM]
    rib = ri_W1[N_DIM:]
    rb1 = ri_b1.reshape(1, 256)
    rb2 = ri_b2.reshape(1, 128)
    rjb1 = rj_b1.reshape(1, 256)
    rjb2 = rj_b2.reshape(1, 128)
    cb1 = c_b1.reshape(1, 128)
    cb2 = c_b2.reshape(1, 1)

    e_t = jnp.pad(E_attr.T, ((0, EP - N_EDGE), (0, 0)))
    srcp = jnp.pad(E[0], (0, EP - N_EDGE), constant_values=N_NODE)
    dstp = jnp.pad(E[1], (0, EP - N_EDGE), constant_values=0)
    h0p = jnp.pad(h_0, ((0, NP - N_NODE), (0, 0)))
    gid = jnp.pad(graph_index, (0, NP - N_NODE),
                  constant_values=N_GRAPH).reshape(1, NP)

    be2 = _edge_bias(e_t, w1e, b1)

    deg = _sc_deg(srcp)[:, :1]
    h = h0p
    z2 = _z_proj(h, w1h)
    for _ in range(T):
        s2 = _sc_step(z2, be2, srcp, dstp)
        h, z2 = _update(s2, deg, h, w2a, w2b, b2,
                        gru_Wih, gru_Whh, bih, bhh, w1h)

    out = _readout(h, h0p, gid, ria, rib, rb1, ri_W2, rb2,
                   rj_W1, rjb1, rj_W2, rjb2, c_W1, cb1, c_W2, cb2)
    return out.reshape(N_GRAPH)


# speedup vs baseline: 1.7910x; 1.0051x over previous
import functools

import jax
import jax.numpy as jnp
from jax import lax
from jax.experimental import pallas as pl
from jax.experimental.pallas import tpu as pltpu
from jax.experimental.pallas import tpu_sc as plsc

N_NODE = 10000
N_EDGE = 320000
N_DIM = 128
E_DIM = 16
M_DIM = 128
G_DIM = 128
T = 3
N_GRAPH = 64

NP = 10240
EP = 327680
NS = 16
NC = 2
C = 40
EPT = EP // NS
NCH = EPT // C
RPT = NP // NS

_f32 = jnp.float32


def _sc_deg_body(srcp, deg_out, src_v, ones_v, zz_v, deg_sh,
                 isem0, isem1):
    c = lax.axis_index("c")
    s = lax.axis_index("s")
    isem = (isem0, isem1)

    for i in range(16):
        for k in range(8):
            zz_v[i, pl.ds(k * 16, 16)] = jnp.zeros((16,), _f32)
    for i in range(C):
        for k in range(8):
            ones_v[i, pl.ds(k * 16, 16)] = jnp.ones((16,), _f32)

    def _zero_d(j, _):
        pltpu.sync_copy(zz_v, deg_sh.at[pl.ds(s * RPT + j * 16, 16)])
        return 0
    lax.fori_loop(0, RPT // 16, _zero_d, 0)
    plsc.subcore_barrier()

    half = EPT // 2
    nchd = half // C
    tbase = s * EPT + c * half
    for b in range(2):
        pltpu.async_copy(srcp.at[pl.ds(tbase + b * C, C)], src_v.at[b],
                         isem[b])

    def _visit(i, _):
        for b in range(2):
            k = i * 2 + b
            pltpu.make_async_copy(srcp.at[pl.ds(0, C)], src_v.at[b],
                                  isem[b]).wait()
            pltpu.sync_copy(ones_v, deg_sh.at[src_v.at[b]], add=True)
            nk = k + 2

            @pl.when(nk < nchd)
            def _():
                pltpu.async_copy(srcp.at[pl.ds(tbase + nk * C, C)],
                                 src_v.at[b], isem[b])
        return 0
    lax.fori_loop(0, nchd // 2, _visit, 0)
    plsc.subcore_barrier()

    pltpu.sync_copy(deg_sh.at[pl.ds(s * RPT, RPT)],
                    deg_out.at[c].at[pl.ds(s * RPT, RPT)])


@functools.cache
def _sc_deg_fn():
    return pl.kernel(
        _sc_deg_body,
        out_type=[jax.ShapeDtypeStruct((NC, NP, 128), _f32)],
        mesh=plsc.VectorSubcoreMesh(core_axis_name="c", subcore_axis_name="s",
                                    num_cores=NC, num_subcores=NS),
        scratch_types=[
            pltpu.VMEM((2, C), jnp.int32),
            pltpu.VMEM((C, 128), _f32),
            pltpu.VMEM((16, 128), _f32),
            pltpu.VMEM_SHARED((NP, 128), _f32),
            pltpu.SemaphoreType.DMA,
            pltpu.SemaphoreType.DMA,
        ],
    )


def _sc_deg(srcp):
    return _sc_deg_fn()(srcp)[0]


def _sc_body(z2, be2, srcp, dstp, s2_out,
             dst_v, src_v, zbuf, bbuf, zrow_v, s_sh,
             gsem0, gsem1, ssem0, ssem1, dsem0, dsem1):
    c = lax.axis_index("c")
    s = lax.axis_index("s")
    gsem = (gsem0, gsem1)
    ssem = (ssem0, ssem1)
    dsem = (dsem0, dsem1)

    for r in range(16):
        for k in range(8):
            zrow_v[r, pl.ds(k * 16, 16)] = jnp.zeros((16,), _f32)

    def _zero_s(j, _):
        pltpu.sync_copy(zrow_v, s_sh.at[pl.ds(s * RPT + j * 16, 16)])
        return 0
    lax.fori_loop(0, RPT // 16, _zero_s, 0)
    plsc.subcore_barrier()

    def _issue_gather(k, b):
        base = s * EPT + k * C
        pltpu.async_copy(z2.at[c].at[dst_v.at[b]], zbuf.at[b], gsem[b])
        pltpu.async_copy(be2.at[c].at[pl.ds(base, C)], bbuf.at[b], gsem[b])

    def _wait_gather(b):
        pltpu.make_async_copy(z2.at[c].at[dst_v.at[b]], zbuf.at[b],
                              gsem[b]).wait()
        pltpu.make_async_copy(be2.at[c].at[pl.ds(0, C)], bbuf.at[b],
                              gsem[b]).wait()

    for b in range(2):
        base = s * EPT + b * C
        pltpu.sync_copy(dstp.at[pl.ds(base, C)], dst_v.at[b])
        _issue_gather(b, b)
        pltpu.async_copy(srcp.at[pl.ds(base, C)], src_v.at[b], ssem[b])

    def _pair(i, _):
        for b in range(2):
            k = i * 2 + b
            nk = k + 2
            _wait_gather(b)
            @pl.when(nk < NCH)
            def _():
                nbase = s * EPT + nk * C
                pltpu.async_copy(dstp.at[pl.ds(nbase, C)], dst_v.at[b],
                                 dsem[b])
            pltpu.make_async_copy(srcp.at[pl.ds(0, C)], src_v.at[b],
                                  ssem[b]).wait()

            @plsc.parallel_loop(0, C, 1, unroll=8)
            def _row(r):
                for kk in range(8):
                    sl = pl.ds(kk * 16, 16)
                    bbuf[b, r, sl] = jnp.maximum(
                        zbuf[b, r, sl] + bbuf[b, r, sl], 0.0)

            @pl.when(nk < NCH)
            def _():
                pltpu.make_async_copy(dstp.at[pl.ds(0, C)], dst_v.at[b],
                                      dsem[b]).wait()
                nbase = s * EPT + nk * C
                pltpu.async_copy(z2.at[c].at[dst_v.at[b]], zbuf.at[b],
                                 gsem[b])

            pltpu.sync_copy(bbuf.at[b], s_sh.at[src_v.at[b]], add=True)

            @pl.when(nk < NCH)
            def _():
                nbase = s * EPT + nk * C
                pltpu.async_copy(srcp.at[pl.ds(nbase, C)], src_v.at[b],
                                 ssem[b])
                pltpu.async_copy(be2.at[c].at[pl.ds(nbase, C)], bbuf.at[b],
                                 gsem[b])
        return 0
    lax.fori_loop(0, NCH // 2, _pair, 0)
    plsc.subcore_barrier()

    pltpu.sync_copy(s_sh.at[pl.ds(s * RPT, RPT)],
                    s2_out.at[c].at[pl.ds(s * RPT, RPT)])


@functools.cache
def _sc_step_fn():
    return pl.kernel(
        _sc_body,
        out_type=[jax.ShapeDtypeStruct((NC, NP, 128), _f32)],
        mesh=plsc.VectorSubcoreMesh(core_axis_name="c", subcore_axis_name="s",
                                    num_cores=NC, num_subcores=NS),
        scratch_types=[
            pltpu.VMEM((2, C), jnp.int32),
            pltpu.VMEM((2, C), jnp.int32),
            pltpu.VMEM((2, C, 128), _f32),
            pltpu.VMEM((2, C, 128), _f32),
            pltpu.VMEM((16, 128), _f32),
            pltpu.VMEM_SHARED((NP, 128), _f32),
            pltpu.SemaphoreType.DMA,
            pltpu.SemaphoreType.DMA,
            pltpu.SemaphoreType.DMA,
            pltpu.SemaphoreType.DMA,
            pltpu.SemaphoreType.DMA,
            pltpu.SemaphoreType.DMA,
        ],
    )


def _sc_step(z2, be2, srcp, dstp):
    return _sc_step_fn()(z2, be2, srcp, dstp)[0]


def _dot_t(a, b):
    return lax.dot_general(a, b, (((1,), (1,)), ((), ())),
                           preferred_element_type=_f32)


def _mm(a, b):
    return jnp.dot(a, b, preferred_element_type=_f32)


def _edge_bias_body(et_ref, w_ref, b_ref, out_ref):
    r = _mm(et_ref[...], w_ref[...]) + b_ref[...]
    out_ref[0] = r[:, :128]
    out_ref[1] = r[:, 128:]


def _edge_bias(e_t, w1e, b1):
    grid = EP // 2048
    return pl.pallas_call(
        _edge_bias_body,
        grid=(grid,),
        in_specs=[pl.BlockSpec((2048, E_DIM), lambda i: (i, 0)),
                  pl.BlockSpec((E_DIM, 256), lambda i: (0, 0)),
                  pl.BlockSpec((1, 256), lambda i: (0, 0))],
        out_specs=pl.BlockSpec((2, 2048, 128), lambda i: (0, i, 0)),
        out_shape=jax.ShapeDtypeStruct((2, EP, 128), _f32),
    )(e_t, w1e, b1)


def _z_proj_body(h_ref, w_ref, out_ref):
    z = _mm(h_ref[...], w_ref[...])
    out_ref[0] = z[:, :128]
    out_ref[1] = z[:, 128:]


def _z_proj(h, w1h):
    grid = NP // 1024
    return pl.pallas_call(
        _z_proj_body,
        grid=(grid,),
        in_specs=[pl.BlockSpec((1024, 128), lambda i: (i, 0)),
                  pl.BlockSpec((128, 256), lambda i: (0, 0))],
        out_specs=pl.BlockSpec((2, 1024, 128), lambda i: (0, i, 0)),
        out_shape=jax.ShapeDtypeStruct((2, NP, 128), _f32),
    )(h, w1h)


def _update_body(s2_ref, deg_ref, h_ref, w2a_ref, w2b_ref, b2_ref,
                 wih_ref, whh_ref, bih_ref, bhh_ref, w1h_ref,
                 h_out, z_out):
    m_v = (_mm(s2_ref[0], w2a_ref[...]) + _mm(s2_ref[1], w2b_ref[...])
           + (deg_ref[0] + deg_ref[1]) * b2_ref[...])
    h = h_ref[...]
    gi = _dot_t(m_v, wih_ref[...]) + bih_ref[...]
    gh = _dot_t(h, whh_ref[...]) + bhh_ref[...]
    r = jax.nn.sigmoid(gi[:, :128] + gh[:, :128])
    z = jax.nn.sigmoid(gi[:, 128:256] + gh[:, 128:256])
    n = jnp.tanh(gi[:, 256:] + r * gh[:, 256:])
    h_new = (1.0 - z) * n + z * h
    h_out[...] = h_new
    zp = _mm(h_new, w1h_ref[...])
    z_out[0] = zp[:, :128]
    z_out[1] = zp[:, 128:]


def _update(s2, deg, h, w2a, w2b, b2, wih, whh, bih, bhh, w1h):
    grid = NP // 1024
    return pl.pallas_call(
        _update_body,
        grid=(grid,),
        in_specs=[pl.BlockSpec((2, 1024, 128), lambda i: (0, i, 0)),
                  pl.BlockSpec((2, 1024, 1), lambda i: (0, i, 0)),
                  pl.BlockSpec((1024, 128), lambda i: (i, 0)),
                  pl.BlockSpec((128, 128), lambda i: (0, 0)),
                  pl.BlockSpec((128, 128), lambda i: (0, 0)),
                  pl.BlockSpec((1, 128), lambda i: (0, 0)),
                  pl.BlockSpec((384, 128), lambda i: (0, 0)),
                  pl.BlockSpec((384, 128), lambda i: (0, 0)),
                  pl.BlockSpec((1, 384), lambda i: (0, 0)),
                  pl.BlockSpec((1, 384), lambda i: (0, 0)),
                  pl.BlockSpec((128, 256), lambda i: (0, 0))],
        out_specs=[pl.BlockSpec((1024, 128), lambda i: (i, 0)),
                   pl.BlockSpec((2, 1024, 128), lambda i: (0, i, 0))],
        out_shape=[jax.ShapeDtypeStruct((NP, 128), _f32),
                   jax.ShapeDtypeStruct((2, NP, 128), _f32)],
    )(s2, deg, h, w2a, w2b, b2, wih, whh, bih, bhh, w1h)


def _readout_body(h_ref, h0_ref, gid_ref,
                  ria_ref, rib_ref, rb1_ref, riw2_ref, rb2_ref,
                  rjw1_ref, rjb1_ref, rjw2_ref, rjb2_ref,
                  cw1_ref, cb1_ref, cw2_ref, cb2_ref,
                  out_ref, racc):
    i = pl.program_id(0)

    @pl.when(i == 0)
    def _():
        racc[...] = jnp.zeros((N_GRAPH, G_DIM), _f32)

    h = h_ref[...]
    h0 = h0_ref[...]
    pre_i = jnp.maximum(_mm(h, ria_ref[...]) + _mm(h0, rib_ref[...])
                        + rb1_ref[...], 0.0)
    i_out = jax.nn.sigmoid(_mm(pre_i, riw2_ref[...]) + rb2_ref[...])
    pre_j = jnp.maximum(_mm(h, rjw1_ref[...]) + rjb1_ref[...], 0.0)
    j_out = _mm(pre_j, rjw2_ref[...]) + rjb2_ref[...]
    p = i_out * j_out

    gid = gid_ref[...]
    seg = lax.broadcasted_iota(jnp.int32, (N_GRAPH, 1024), 0)
    ht = (seg == gid).astype(_f32)
    racc[...] += lax.dot_general(ht, p, (((1,), (0,)), ((), ())),
                                 preferred_element_type=_f32)

    @pl.when(i == pl.num_programs(0) - 1)
    def _():
        rr = racc[...]
        l1 = jnp.maximum(_mm(rr, cw1_ref[...]) + cb1_ref[...], 0.0)
        logit = _mm(l1, cw2_ref[...]) + cb2_ref[...]
        out_ref[...] = jax.nn.sigmoid(logit)


def _readout(h, h0, gid, ria, rib, rb1, riw2, rb2,
             rjw1, rjb1, rjw2, rjb2, cw1, cb1, cw2, cb2):
    grid = NP // 1024
    return pl.pallas_call(
        _readout_body,
        grid=(grid,),
        in_specs=[pl.BlockSpec((1024, 128), lambda i: (i, 0)),
                  pl.BlockSpec((1024, 128), lambda i: (i, 0)),
                  pl.BlockSpec((1, 1024), lambda i: (0, i)),
                  pl.BlockSpec((128, 256), lambda i: (0, 0)),
                  pl.BlockSpec((128, 256), lambda i: (0, 0)),
                  pl.BlockSpec((1, 256), lambda i: (0, 0)),
                  pl.BlockSpec((256, 128), lambda i: (0, 0)),
                  pl.BlockSpec((1, 128), lambda i: (0, 0)),
                  pl.BlockSpec((128, 256), lambda i: (0, 0)),
                  pl.BlockSpec((1, 256), lambda i: (0, 0)),
                  pl.BlockSpec((256, 128), lambda i: (0, 0)),
                  pl.BlockSpec((1, 128), lambda i: (0, 0)),
                  pl.BlockSpec((128, 128), lambda i: (0, 0)),
                  pl.BlockSpec((1, 128), lambda i: (0, 0)),
                  pl.BlockSpec((128, 1), lambda i: (0, 0)),
                  pl.BlockSpec((1, 1), lambda i: (0, 0))],
        out_specs=pl.BlockSpec((N_GRAPH, 1), lambda i: (0, 0)),
        out_shape=jax.ShapeDtypeStruct((N_GRAPH, 1), _f32),
        scratch_shapes=[pltpu.VMEM((N_GRAPH, G_DIM), _f32)],
        compiler_params=pltpu.CompilerParams(
            dimension_semantics=("arbitrary",)),
    )(h, h0, gid, ria, rib, rb1, riw2, rb2,
      rjw1, rjb1, rjw2, rjb2, cw1, cb1, cw2, cb2)


def kernel(h_0, E_attr, m_W1, m_b1, m_W2, m_b2, gru_Wih, gru_Whh, gru_bih,
           gru_bhh, ri_W1, ri_b1, ri_W2, ri_b2, rj_W1, rj_b1, rj_W2, rj_b2,
           c_W1, c_b1, c_W2, c_b2, graph_index, E):
    w1h = m_W1[:N_DIM]
    w1e = m_W1[N_DIM:]
    w2a = m_W2[:128]
    w2b = m_W2[128:]
    b1 = m_b1.reshape(1, 256)
    b2 = m_b2.reshape(1, 128)
    bih = gru_bih.reshape(1, 384)
    bhh = gru_bhh.reshape(1, 384)
    ria = ri_W1[:N_DIM]
    rib = ri_W1[N_DIM:]
    rb1 = ri_b1.reshape(1, 256)
    rb2 = ri_b2.reshape(1, 128)
    rjb1 = rj_b1.reshape(1, 256)
    rjb2 = rj_b2.reshape(1, 128)
    cb1 = c_b1.reshape(1, 128)
    cb2 = c_b2.reshape(1, 1)

    e_t = jnp.pad(E_attr.T, ((0, EP - N_EDGE), (0, 0)))
    srcp = jnp.pad(E[0], (0, EP - N_EDGE), constant_values=N_NODE)
    dstp = jnp.pad(E[1], (0, EP - N_EDGE), constant_values=0)
    h0p = jnp.pad(h_0, ((0, NP - N_NODE), (0, 0)))
    gid = jnp.pad(graph_index, (0, NP - N_NODE),
                  constant_values=N_GRAPH).reshape(1, NP)

    be2 = _edge_bias(e_t, w1e, b1)

    deg = _sc_deg(srcp)[:, :, :1]
    h = h0p
    z2 = _z_proj(h, w1h)
    for _ in range(T):
        s2 = _sc_step(z2, be2, srcp, dstp)
        h, z2 = _update(s2, deg, h, w2a, w2b, b2,
                        gru_Wih, gru_Whh, bih, bhh, w1h)

    out = _readout(h, h0p, gid, ria, rib, rb1, ri_W2, rb2,
                   rj_W1, rjb1, rj_W2, rjb2, c_W1, cb1, c_W2, cb2)
    return out.reshape(N_GRAPH)
